# baseline (speedup 1.0000x reference)
import functools

import jax
import jax.numpy as jnp
from jax import lax
from jax.experimental import pallas as pl
from jax.experimental.pallas import tpu as pltpu
from jax.experimental.pallas import tpu_sc as plsc

_DIM = 768
_INTER = 256
_E = 64
_T = 2048
_BT = 64
_NT = _T // _BT + 63
_TP = _NT * _BT
_NC = 2
_NS = 16
_NW = _NC * _NS
_CHUNK = _T // _NW


def _silu(v):
    return v * jax.nn.sigmoid(v)


def _k1_body(x_ref, wg_ref, bg_ref, pos_ref, w_ref, te_ref):
    xf = x_ref[...]
    wg = wg_ref[...]
    logits = lax.dot_general(xf, wg, (((1,), (1,)), ((), ())),
                             preferred_element_type=jnp.float32)
    m0 = jnp.max(logits, axis=1, keepdims=True)
    ex = jnp.exp(logits - m0)
    scores = ex / jnp.sum(ex, axis=1, keepdims=True)
    sb = scores + bg_ref[...]
    mx = jnp.max(sb, axis=1, keepdims=True)
    iota_e = lax.broadcasted_iota(jnp.int32, (_T, _E), 1)
    eidx = jnp.min(jnp.where(sb == mx, iota_e, _E), axis=1, keepdims=True)
    onehot = (iota_e == eidx).astype(jnp.float32)
    w_tok = jnp.sum(scores * onehot, axis=1, keepdims=True)

    ch = 128
    l_strict = (lax.broadcasted_iota(jnp.int32, (ch, ch), 1)
                < lax.broadcasted_iota(jnp.int32, (ch, ch), 0)).astype(jnp.float32)
    base = jnp.zeros((1, _E), dtype=jnp.float32)
    rank_rows = []
    for c in range(_T // ch):
        chunk = lax.slice_in_dim(onehot, c * ch, (c + 1) * ch, axis=0)
        r = lax.dot_general(l_strict, chunk, (((1,), (0,)), ((), ())),
                            preferred_element_type=jnp.float32)
        rank_rows.append(r + base)
        base = base + jnp.sum(chunk, axis=0, keepdims=True)
    rank = jnp.concatenate(rank_rows, axis=0)
    counts = base

    tiles = jnp.floor((counts + (_BT - 1)) * (1.0 / _BT))
    l64 = (lax.broadcasted_iota(jnp.int32, (_E, _E), 0)
           < lax.broadcasted_iota(jnp.int32, (_E, _E), 1)).astype(jnp.float32)
    tile_start = lax.dot_general(tiles, l64, (((1,), (0,)), ((), ())),
                                 preferred_element_type=jnp.float32)

    ts_tok = jnp.sum(onehot * tile_start, axis=1, keepdims=True)
    rk_tok = jnp.sum(onehot * rank, axis=1, keepdims=True)
    pos_ref[...] = (ts_tok * _BT + rk_tok).astype(jnp.int32)
    w_ref[...] = w_tok

    te_ref[...] = jnp.concatenate([tile_start, tiles], axis=0).astype(jnp.int32)


def _k1(x2d, Wg, bg):
    return pl.pallas_call(
        _k1_body,
        out_shape=(
            jax.ShapeDtypeStruct((_T, 1), jnp.int32),
            jax.ShapeDtypeStruct((_T, 1), jnp.float32),
            jax.ShapeDtypeStruct((2, _E), jnp.int32),
        ),
    )(x2d, Wg, bg.reshape(1, _E))


def _sc_mesh():
    return plsc.VectorSubcoreMesh(core_axis_name="c", subcore_axis_name="s",
                                  num_cores=_NC, num_subcores=_NS)


def _k2_body(pos_hbm, x_hbm, xs_hbm, idx_v, rows_v, sem):
    wid = lax.axis_index("s") * _NC + lax.axis_index("c")
    base = wid * _CHUNK
    pltpu.sync_copy(pos_hbm.at[pl.ds(base, _CHUNK)], idx_v)
    pltpu.sync_copy(x_hbm.at[pl.ds(base, _CHUNK), :], rows_v)
    pltpu.async_copy(rows_v, xs_hbm.at[idx_v], sem).wait()


def _k2(pos, x2d):
    return pl.kernel(
        _k2_body,
        out_type=jax.ShapeDtypeStruct((_TP, _DIM), jnp.float32),
        mesh=_sc_mesh(),
        scratch_types=[
            pltpu.VMEM((_CHUNK,), jnp.int32),
            pltpu.VMEM((_CHUNK, _DIM), jnp.float32),
            pltpu.SemaphoreType.DMA,
        ],
    )(pos, x2d)


_EPG = 4


def _k3_body(te_ref, xs_ref, w1_ref, w3_ref, w2_ref, out_ref):
    step = pl.program_id(0)
    for k in range(_EPG):
        e = step * _EPG + k
        ts = te_ref[0, e]
        nt = te_ref[1, e]
        w1 = w1_ref[k].astype(jnp.bfloat16)
        w3 = w3_ref[k].astype(jnp.bfloat16)
        w2 = w2_ref[k].astype(jnp.bfloat16)

        def body(j, carry):
            r0 = (ts + j) * _BT
            xb = xs_ref[pl.ds(r0, _BT), :].astype(jnp.bfloat16)
            a = lax.dot_general(xb, w1, (((1,), (1,)), ((), ())), preferred_element_type=jnp.float32)
            b = lax.dot_general(xb, w3, (((1,), (1,)), ((), ())), preferred_element_type=jnp.float32)
            h = (_silu(a) * b).astype(jnp.bfloat16)
            out_ref[pl.ds(r0, _BT), :] = lax.dot_general(
                h, w2, (((1,), (1,)), ((), ())), preferred_element_type=jnp.float32)
            return carry

        lax.fori_loop(0, nt, body, 0)


def _k3(xs, te, W1, W2, W3):
    grid_spec = pltpu.PrefetchScalarGridSpec(
        num_scalar_prefetch=1,
        grid=(_E // _EPG,),
        in_specs=[
            pl.BlockSpec((_TP, _DIM), lambda e, te: (0, 0)),
            pl.BlockSpec((_EPG, _INTER, _DIM), lambda e, te: (e, 0, 0)),
            pl.BlockSpec((_EPG, _INTER, _DIM), lambda e, te: (e, 0, 0)),
            pl.BlockSpec((_EPG, _DIM, _INTER), lambda e, te: (e, 0, 0)),
        ],
        out_specs=pl.BlockSpec((_TP, _DIM), lambda e, te: (0, 0)),
    )
    return pl.pallas_call(
        _k3_body,
        grid_spec=grid_spec,
        out_shape=jax.ShapeDtypeStruct((_TP, _DIM), jnp.float32),
    )(te, xs, W1, W3, W2)


def _k4_body(pos_hbm, ys_hbm, yr_hbm, idx_v, rows_v, sem):
    wid = lax.axis_index("s") * _NC + lax.axis_index("c")
    base = wid * _CHUNK
    pltpu.sync_copy(pos_hbm.at[pl.ds(base, _CHUNK)], idx_v)
    pltpu.async_copy(ys_hbm.at[idx_v], rows_v, sem).wait()
    pltpu.sync_copy(rows_v, yr_hbm.at[pl.ds(base, _CHUNK), :])


def _k4(pos, ys):
    return pl.kernel(
        _k4_body,
        out_type=jax.ShapeDtypeStruct((_T, _DIM), jnp.float32),
        mesh=_sc_mesh(),
        scratch_types=[
            pltpu.VMEM((_CHUNK,), jnp.int32),
            pltpu.VMEM((_CHUNK, _DIM), jnp.float32),
            pltpu.SemaphoreType.DMA,
        ],
    )(pos, ys)


def _kz_body(x_ref, ws1_ref, ws3_ref, ws2_ref, out_ref):
    xb = x_ref[...].astype(jnp.bfloat16)
    ws1 = ws1_ref[...].astype(jnp.bfloat16)
    ws3 = ws3_ref[...].astype(jnp.bfloat16)
    a = lax.dot_general(xb, ws1, (((1,), (1,)), ((), ())), preferred_element_type=jnp.float32)
    b = lax.dot_general(xb, ws3, (((1,), (1,)), ((), ())), preferred_element_type=jnp.float32)
    h = (_silu(a) * b).astype(jnp.bfloat16)
    out_ref[...] = lax.dot_general(h, ws2_ref[...].astype(jnp.bfloat16), (((1,), (1,)), ((), ())),
                                   preferred_element_type=jnp.float32)


def _kz(x2d, Ws1, Ws2, Ws3):
    bc = 256
    return pl.pallas_call(
        _kz_body,
        grid=(_T // bc,),
        in_specs=[
            pl.BlockSpec((bc, _DIM), lambda i: (i, 0)),
            pl.BlockSpec((_INTER, _DIM), lambda i: (0, 0)),
            pl.BlockSpec((_INTER, _DIM), lambda i: (0, 0)),
            pl.BlockSpec((_DIM, _INTER), lambda i: (0, 0)),
        ],
        out_specs=pl.BlockSpec((bc, _DIM), lambda i: (i, 0)),
        out_shape=jax.ShapeDtypeStruct((_T, _DIM), jnp.float32),
    )(x2d, Ws1, Ws3, Ws2)


def _k5_body(z_ref, yr_ref, w_ref, out_ref):
    out_ref[...] = w_ref[...] * yr_ref[...] + z_ref[...]


def _k5(z, yr, w):
    bc = 512
    return pl.pallas_call(
        _k5_body,
        grid=(_T // bc,),
        in_specs=[
            pl.BlockSpec((bc, _DIM), lambda i: (i, 0)),
            pl.BlockSpec((bc, _DIM), lambda i: (i, 0)),
            pl.BlockSpec((bc, 1), lambda i: (i, 0)),
        ],
        out_specs=pl.BlockSpec((bc, _DIM), lambda i: (i, 0)),
        out_shape=jax.ShapeDtypeStruct((_T, _DIM), jnp.float32),
    )(z, yr, w)


def kernel(x, Wg, bg, W1, W2, W3, Ws1, Ws2, Ws3):
    shape = x.shape
    x2d = x.reshape(_T, _DIM)
    pos2d, w2d, te = _k1(x2d, Wg, bg)
    pos = pos2d.reshape(_T)
    xs = _k2(pos, x2d)
    z = _kz(x2d, Ws1, Ws2, Ws3)
    ys = _k3(xs, te, W1, W2, W3)
    yr = _k4(pos, ys)
    y = _k5(z, yr, w2d)
    return y.reshape(shape)


# speedup vs baseline: 1.0096x; 1.0096x over previous
import jax
import jax.numpy as jnp
from jax import lax
from jax.experimental import pallas as pl
from jax.experimental.pallas import tpu as pltpu
from jax.experimental.pallas import tpu_sc as plsc

_DIM = 768
_INTER = 256
_E = 64
_T = 2048
_BT = 64
_NT = _T // _BT + 63
_TP = _NT * _BT
_NC = 2
_NS = 16
_NW = _NC * _NS
_CHUNK = _T // _NW


def _silu(v):
    return v * jax.nn.sigmoid(v)


def _k1_body(x_ref, wg_ref, bg_ref, pos_ref, w_ref, te_ref):
    xf = x_ref[...]
    wg = wg_ref[...]
    logits = lax.dot_general(xf, wg, (((1,), (1,)), ((), ())),
                             preferred_element_type=jnp.float32)
    m0 = jnp.max(logits, axis=1, keepdims=True)
    ex = jnp.exp(logits - m0)
    scores = ex / jnp.sum(ex, axis=1, keepdims=True)
    sb = scores + bg_ref[...]
    mx = jnp.max(sb, axis=1, keepdims=True)
    iota_e = lax.broadcasted_iota(jnp.int32, (_T, _E), 1)
    eidx = jnp.min(jnp.where(sb == mx, iota_e, _E), axis=1, keepdims=True)
    onehot = (iota_e == eidx).astype(jnp.float32)
    w_tok = jnp.sum(scores * onehot, axis=1, keepdims=True)

    ch = 128
    l_strict = (lax.broadcasted_iota(jnp.int32, (ch, ch), 1)
                < lax.broadcasted_iota(jnp.int32, (ch, ch), 0)).astype(jnp.float32)
    base = jnp.zeros((1, _E), dtype=jnp.float32)
    rank_rows = []
    for c in range(_T // ch):
        chunk = lax.slice_in_dim(onehot, c * ch, (c + 1) * ch, axis=0)
        r = lax.dot_general(l_strict, chunk, (((1,), (0,)), ((), ())),
                            preferred_element_type=jnp.float32)
        rank_rows.append(r + base)
        base = base + jnp.sum(chunk, axis=0, keepdims=True)
    rank = jnp.concatenate(rank_rows, axis=0)
    counts = base

    tiles = jnp.floor((counts + (_BT - 1)) * (1.0 / _BT))
    l64 = (lax.broadcasted_iota(jnp.int32, (_E, _E), 0)
           < lax.broadcasted_iota(jnp.int32, (_E, _E), 1)).astype(jnp.float32)
    tile_start = lax.dot_general(tiles, l64, (((1,), (0,)), ((), ())),
                                 preferred_element_type=jnp.float32)

    ts_tok = jnp.sum(onehot * tile_start, axis=1, keepdims=True)
    rk_tok = jnp.sum(onehot * rank, axis=1, keepdims=True)
    pos_ref[...] = (ts_tok * _BT + rk_tok).astype(jnp.int32)
    w_ref[...] = w_tok

    te_ref[...] = jnp.concatenate([tile_start, tiles], axis=0).astype(jnp.int32)


def _k1(x2d, Wg, bg):
    return pl.pallas_call(
        _k1_body,
        out_shape=(
            jax.ShapeDtypeStruct((_T, 1), jnp.int32),
            jax.ShapeDtypeStruct((_T, 1), jnp.float32),
            jax.ShapeDtypeStruct((2, _E), jnp.int32),
        ),
    )(x2d, Wg, bg.reshape(1, _E))


def _sc_mesh():
    return plsc.VectorSubcoreMesh(core_axis_name="c", subcore_axis_name="s",
                                  num_cores=_NC, num_subcores=_NS)


def _k2_body(pos_hbm, x_hbm, xs_hbm, idx_v, rows_v, sem):
    wid = lax.axis_index("s") * _NC + lax.axis_index("c")
    base = wid * _CHUNK
    pltpu.sync_copy(pos_hbm.at[pl.ds(base, _CHUNK)], idx_v)
    pltpu.sync_copy(x_hbm.at[pl.ds(base, _CHUNK), :], rows_v)
    pltpu.async_copy(rows_v, xs_hbm.at[idx_v], sem).wait()


def _k2(pos, x2d):
    return pl.kernel(
        _k2_body,
        out_type=jax.ShapeDtypeStruct((_TP, _DIM), jnp.float32),
        mesh=_sc_mesh(),
        scratch_types=[
            pltpu.VMEM((_CHUNK,), jnp.int32),
            pltpu.VMEM((_CHUNK, _DIM), jnp.float32),
            pltpu.SemaphoreType.DMA,
        ],
    )(pos, x2d)


_EPG = 4


def _k3_body(te_ref, xs_ref, w1_ref, w3_ref, w2_ref, out_ref):
    step = pl.program_id(0)
    for k in range(_EPG):
        e = step * _EPG + k
        ts = te_ref[0, e]
        nt = te_ref[1, e]
        w1 = w1_ref[k].astype(jnp.bfloat16)
        w3 = w3_ref[k].astype(jnp.bfloat16)
        w2 = w2_ref[k].astype(jnp.bfloat16)

        def body(j, carry):
            r0 = (ts + j) * _BT
            xb = xs_ref[pl.ds(r0, _BT), :].astype(jnp.bfloat16)
            a = lax.dot_general(xb, w1, (((1,), (1,)), ((), ())), preferred_element_type=jnp.float32)
            b = lax.dot_general(xb, w3, (((1,), (1,)), ((), ())), preferred_element_type=jnp.float32)
            h = (_silu(a) * b).astype(jnp.bfloat16)
            out_ref[pl.ds(r0, _BT), :] = lax.dot_general(
                h, w2, (((1,), (1,)), ((), ())), preferred_element_type=jnp.float32)
            return carry

        lax.fori_loop(0, nt, body, 0)


def _k3(xs, te, W1, W2, W3):
    grid_spec = pltpu.PrefetchScalarGridSpec(
        num_scalar_prefetch=1,
        grid=(_E // _EPG,),
        in_specs=[
            pl.BlockSpec((_TP, _DIM), lambda e, te: (0, 0)),
            pl.BlockSpec((_EPG, _INTER, _DIM), lambda e, te: (e, 0, 0)),
            pl.BlockSpec((_EPG, _INTER, _DIM), lambda e, te: (e, 0, 0)),
            pl.BlockSpec((_EPG, _DIM, _INTER), lambda e, te: (e, 0, 0)),
        ],
        out_specs=pl.BlockSpec((_TP, _DIM), lambda e, te: (0, 0)),
    )
    return pl.pallas_call(
        _k3_body,
        grid_spec=grid_spec,
        out_shape=jax.ShapeDtypeStruct((_TP, _DIM), jnp.float32),
    )(te, xs, W1, W3, W2)


def _k4_body(pos_hbm, ys_hbm, yr_hbm, idx_v, rows_v, sem):
    wid = lax.axis_index("s") * _NC + lax.axis_index("c")
    base = wid * _CHUNK
    pltpu.sync_copy(pos_hbm.at[pl.ds(base, _CHUNK)], idx_v)
    pltpu.async_copy(ys_hbm.at[idx_v], rows_v, sem).wait()
    pltpu.sync_copy(rows_v, yr_hbm.at[pl.ds(base, _CHUNK), :])


def _k4(pos, ys):
    return pl.kernel(
        _k4_body,
        out_type=jax.ShapeDtypeStruct((_T, _DIM), jnp.float32),
        mesh=_sc_mesh(),
        scratch_types=[
            pltpu.VMEM((_CHUNK,), jnp.int32),
            pltpu.VMEM((_CHUNK, _DIM), jnp.float32),
            pltpu.SemaphoreType.DMA,
        ],
    )(pos, ys)


def _k5_body(x_ref, yr_ref, w_ref, ws1_ref, ws3_ref, ws2_ref, out_ref):
    xb = x_ref[...].astype(jnp.bfloat16)
    ws1 = ws1_ref[...].astype(jnp.bfloat16)
    ws3 = ws3_ref[...].astype(jnp.bfloat16)
    a = lax.dot_general(xb, ws1, (((1,), (1,)), ((), ())), preferred_element_type=jnp.float32)
    b = lax.dot_general(xb, ws3, (((1,), (1,)), ((), ())), preferred_element_type=jnp.float32)
    h = (_silu(a) * b).astype(jnp.bfloat16)
    z = lax.dot_general(h, ws2_ref[...].astype(jnp.bfloat16), (((1,), (1,)), ((), ())),
                        preferred_element_type=jnp.float32)
    out_ref[...] = w_ref[...] * yr_ref[...] + z


def _k5(x2d, yr, w, Ws1, Ws2, Ws3):
    bc = 256
    return pl.pallas_call(
        _k5_body,
        grid=(_T // bc,),
        in_specs=[
            pl.BlockSpec((bc, _DIM), lambda i: (i, 0)),
            pl.BlockSpec((bc, _DIM), lambda i: (i, 0)),
            pl.BlockSpec((bc, 1), lambda i: (i, 0)),
            pl.BlockSpec((_INTER, _DIM), lambda i: (0, 0)),
            pl.BlockSpec((_INTER, _DIM), lambda i: (0, 0)),
            pl.BlockSpec((_DIM, _INTER), lambda i: (0, 0)),
        ],
        out_specs=pl.BlockSpec((bc, _DIM), lambda i: (i, 0)),
        out_shape=jax.ShapeDtypeStruct((_T, _DIM), jnp.float32),
    )(x2d, yr, w, Ws1, Ws3, Ws2)


def kernel(x, Wg, bg, W1, W2, W3, Ws1, Ws2, Ws3):
    shape = x.shape
    x2d = x.reshape(_T, _DIM)
    pos2d, w2d, te = _k1(x2d, Wg, bg)
    pos = pos2d.reshape(_T)
    xs = _k2(pos, x2d)
    ys = _k3(xs, te, W1, W2, W3)
    yr = _k4(pos, ys)
    y = _k5(x2d, yr, w2d, Ws1, Ws2, Ws3)
    return y.reshape(shape)
